# baseline (speedup 1.0000x reference)
import jax
import jax.numpy as jnp
from jax.experimental import pallas as pl
from jax.experimental.pallas import tpu as pltpu

_NUM_CLASSES = 100000
_FEAT_LEN = 64
_BATCH = 1024
_BN = 4096
_NTILES = _NUM_CLASSES // _BN
_NSTEPS = _NTILES // 2
_TAIL = _NUM_CLASSES - _NTILES * _BN
_TAIL_COL = _NTILES * _BN
_NSPLIT = 8
_RB = _BATCH // _NSPLIT


def _copies(src, hbm_out, sems, col, width):
    return [
        pltpu.make_async_copy(
            src.at[pl.ds(r * _RB, _RB), :],
            hbm_out.at[pl.ds(r * _RB, _RB), pl.ds(col, width)],
            sems.at[r],
        )
        for r in range(_NSPLIT)
    ]


def _mm_kernel(feats_ref, pid_ref, lut_a_ref, lut_b_ref, lut_tail_ref,
               hbm_out, labels_hbm, scratch0, scratch1, tail_scr, labels_scr,
               sems0, sems1, tsems, lsem):
    j = pl.program_id(0)
    f = feats_ref[...].astype(jnp.bfloat16)

    @pl.when(j > 0)
    def _wait_prev_labels():
        pltpu.make_async_copy(labels_scr, labels_hbm, lsem).wait()

    p = pid_ref[...]
    labels_scr[...] = jnp.where((p < 0) | (p >= _NUM_CLASSES), -1, p)
    pltpu.make_async_copy(labels_scr, labels_hbm, lsem).start()

    @pl.when(j > 0)
    def _wait_prev_a():
        for c in _copies(scratch0, hbm_out, sems0, (2 * j - 2) * _BN, _BN):
            c.wait()

    w_a = lut_a_ref[...].astype(jnp.bfloat16)
    scratch0[...] = jax.lax.dot_general(
        f, w_a, (((1,), (1,)), ((), ())), preferred_element_type=jnp.float32
    )
    for c in _copies(scratch0, hbm_out, sems0, (2 * j) * _BN, _BN):
        c.start()

    @pl.when(j == 0)
    def _tail():
        wt = lut_tail_ref[...].astype(jnp.bfloat16)
        tail_scr[...] = jax.lax.dot_general(
            f, wt, (((1,), (1,)), ((), ())), preferred_element_type=jnp.float32
        )
        for c in _copies(tail_scr, hbm_out, tsems, _TAIL_COL, _TAIL):
            c.start()

    @pl.when(j > 0)
    def _wait_prev_b():
        for c in _copies(scratch1, hbm_out, sems1, (2 * j - 1) * _BN, _BN):
            c.wait()

    w_b = lut_b_ref[...].astype(jnp.bfloat16)
    scratch1[...] = jax.lax.dot_general(
        f, w_b, (((1,), (1,)), ((), ())), preferred_element_type=jnp.float32
    )
    for c in _copies(scratch1, hbm_out, sems1, (2 * j + 1) * _BN, _BN):
        c.start()

    @pl.when(j == _NSTEPS - 1)
    def _wait_last():
        for c in _copies(scratch0, hbm_out, sems0, (2 * j) * _BN, _BN):
            c.wait()
        for c in _copies(scratch1, hbm_out, sems1, (2 * j + 1) * _BN, _BN):
            c.wait()
        for c in _copies(tail_scr, hbm_out, tsems, _TAIL_COL, _TAIL):
            c.wait()
        pltpu.make_async_copy(labels_scr, labels_hbm, lsem).wait()


def kernel(feats, pid_labels, lookup_table):
    pid2d = pid_labels.reshape(8, 128)
    lut_tail = lookup_table[_TAIL_COL:, :]
    score, labels2d = pl.pallas_call(
        _mm_kernel,
        grid=(_NSTEPS,),
        in_specs=[
            pl.BlockSpec((_BATCH, _FEAT_LEN), lambda j: (0, 0)),
            pl.BlockSpec((8, 128), lambda j: (0, 0)),
            pl.BlockSpec((_BN, _FEAT_LEN), lambda j: (2 * j, 0)),
            pl.BlockSpec((_BN, _FEAT_LEN), lambda j: (2 * j + 1, 0)),
            pl.BlockSpec((_TAIL, _FEAT_LEN), lambda j: (0, 0)),
        ],
        out_specs=[
            pl.BlockSpec(memory_space=pltpu.MemorySpace.HBM),
            pl.BlockSpec(memory_space=pltpu.MemorySpace.HBM),
        ],
        out_shape=[
            jax.ShapeDtypeStruct((_BATCH, _NUM_CLASSES), jnp.float32),
            jax.ShapeDtypeStruct((8, 128), jnp.int32),
        ],
        scratch_shapes=[
            pltpu.VMEM((_BATCH, _BN), jnp.float32),
            pltpu.VMEM((_BATCH, _BN), jnp.float32),
            pltpu.VMEM((_BATCH, _TAIL), jnp.float32),
            pltpu.VMEM((8, 128), jnp.int32),
            pltpu.SemaphoreType.DMA((_NSPLIT,)),
            pltpu.SemaphoreType.DMA((_NSPLIT,)),
            pltpu.SemaphoreType.DMA((_NSPLIT,)),
            pltpu.SemaphoreType.DMA(()),
        ],
        compiler_params=pltpu.CompilerParams(
            dimension_semantics=("arbitrary",),
        ),
    )(feats, pid2d, lookup_table, lookup_table, lut_tail)
    return (score, labels2d.reshape(-1))
